# baseline (speedup 1.0000x reference)
import functools

import jax
import jax.numpy as jnp
from jax.experimental import pallas as pl
from jax.experimental.pallas import tpu as pltpu

_EPS = 1e-5
_NEG = -1e30


def _fold_bn(p):
    s = p["gamma"] / jnp.sqrt(p["var"] + _EPS)
    t = p["beta"] - p["mean"] * s
    return s, t


def _dotT(a, b):
    return jax.lax.dot_general(
        a, b, (((1,), (1,)), ((), ())), preferred_element_type=jnp.float32
    )


def _dot(a, b):
    return jnp.dot(a, b, preferred_element_type=jnp.float32)


def _bdot(a, b):
    return jnp.dot(a.astype(jnp.bfloat16), b, preferred_element_type=jnp.float32)


def _gnn_kernel(
    x_ref, x3t_ref,
    e1n_ref, e1c_ref, s1e_ref, t1e_ref,
    e2n_ref, e2c_ref, s2e_ref, t2e_ref,
    ac1_ref, scb_ref, tcb_ref,
    ac2_ref, bc2_ref,
    ae_ref, se_ref, te_ref,
    ar_ref, sr_ref, tr_ref,
    s1_ref, tsc1_ref,
    w1_ref, bsc1_ref,
    w2_ref, bsc2_ref,
    s2_ref, tsc2_ref,
    out_ref,
    acc_ref,
    *, n_chunks, pts, k_sel,
):
    cp = pts
    gpc = cp // 16
    lane_i = jax.lax.broadcasted_iota(jnp.int32, (cp, cp), 1)
    lane16 = lane_i % 16
    in_group = (jax.lax.broadcasted_iota(jnp.int32, (cp, cp), 0) // 16) == (lane_i // 16)

    def load_stage(c):
        xt = x_ref[pl.ds(c * cp, cp), :]
        x3 = xt[:, 0:3]
        x3t = x3t_ref[:, pl.ds(c * cp, cp)]

        x3b = x3.astype(jnp.bfloat16)
        inner = -2.0 * _dotT(x3b, x3b)
        xx_row = jnp.sum(x3t * x3t, axis=0, keepdims=True)
        xx_col = jnp.sum(x3 * x3, axis=1, keepdims=True)
        pd = (-xx_row) - inner - xx_col
        return xt, pd

    def rank_stage(pd):
        rank = jnp.zeros((cp, cp), jnp.float32)
        for s in range(1, 16):
            wrap = lane16 + s >= 16
            rot = jnp.where(
                wrap,
                jnp.roll(pd, 16 - s, axis=1),
                jnp.roll(pd, -s, axis=1),
            )
            gt = (rot > pd) | ((rot == pd) & wrap)
            rank = rank + gt.astype(jnp.float32)
        return jnp.where(in_group, rank, 16.0)

    def edge_layer(rank, xin, wn, wc, srow, trow):
        vc = _bdot(xin, wc)
        cmx = cmn = None
        for j in range(k_sel):
            p = (rank == float(j)).astype(jnp.float32)
            g = _dot(p, xin)
            h = _bdot(g - xin, wn)
            if cmx is None:
                cmx = cmn = h
            else:
                cmx = jnp.maximum(cmx, h)
                cmn = jnp.minimum(cmn, h)
        hsel = jnp.where(srow >= 0.0, cmx + vc, cmn + vc)
        return jnp.maximum(hsel * srow + trow, 0.0)

    def tail_stage(cs, ranks, x1s):
        x2s = [
            edge_layer(rk, x1, e2n_ref[:], e2c_ref[:], s2e_ref[:], t2e_ref[:])
            for rk, x1 in zip(ranks, x1s)
        ]
        xcats = [jnp.concatenate([x1, x2], axis=1) for x1, x2 in zip(x1s, x2s)]
        c1s = [
            jnp.maximum(_bdot(xc, ac1_ref[:]) * scb_ref[:] + tcb_ref[:], 0.0)
            for xc in xcats
        ]
        c2s = [_bdot(c1, ac2_ref[:]) + bc2_ref[:] for c1 in c1s]
        xgs = [jax.nn.sigmoid(c2) * xc for c2, xc in zip(c2s, xcats)]
        es = [
            jnp.maximum(_bdot(xg, ae_ref[:]) * se_ref[:] + te_ref[:], 0.0)
            for xg in xgs
        ]
        for c, e in zip(cs, es):
            m = jnp.max(e.reshape(gpc, 16, e.shape[1]), axis=1)
            acc_ref[pl.ds(c * gpc, gpc), :] = m

    nu = 16

    def chunk_body(c, _):
        cs = [nu * c + u for u in range(nu)]
        loaded = [load_stage(ci) for ci in cs]
        ranks = [rank_stage(pd) for _, pd in loaded]
        x1s = [
            edge_layer(rk, xt, e1n_ref[:], e1c_ref[:], s1e_ref[:], t1e_ref[:])
            for rk, (xt, _) in zip(ranks, loaded)
        ]
        nt = 16
        for t in range(0, nu, nt):
            tail_stage(cs[t : t + nt], ranks[t : t + nt], x1s[t : t + nt])
        return 0

    jax.lax.fori_loop(0, n_chunks // nu, chunk_body, 0)

    av = acc_ref[:]
    r = jnp.maximum(_bdot(av, ar_ref[:]) * sr_ref[:] + tr_ref[:], 0.0)
    xb = (r + r) * s1_ref[:] + tsc1_ref[:]
    h = jnp.maximum(_bdot(xb, w1_ref[:]) + bsc1_ref[:], 0.0)
    h2 = _bdot(h, w2_ref[:]) + bsc2_ref[:]
    out_ref[:] = (xb + h2) * s2_ref[:] + tsc2_ref[:]


def kernel(xyz, feats, params):
    B, M, K, _ = xyz.shape
    bm = B * M
    x = jnp.concatenate([xyz, feats], axis=-1).reshape(bm * K, -1)
    cin = x.shape[1]
    x3t = xyz.reshape(bm * K, 3).T

    bf16 = jnp.bfloat16

    def row(v):
        return v.reshape(1, -1)

    ops = []

    def edge_prep(lyr):
        w = lyr["w"]
        c = w.shape[1] // 2
        s, t = _fold_bn(lyr["bn"])
        wn = w[:, :c].T.astype(bf16)
        wc = w[:, c:].T.astype(bf16)
        return wn, wc, row(s), row(t)

    ops += list(edge_prep(params["edge"][0]))
    ops += list(edge_prep(params["edge"][1]))

    s, t = _fold_bn(params["calib_bn"])
    ops += [params["calib_w1"].T.astype(bf16), row(s), row(t)]
    ops += [params["calib_w2"].T.astype(bf16), row(params["calib_b2"])]

    s, t = _fold_bn(params["exp_bn"])
    ops += [params["exp_w"].T.astype(bf16), row(s), row(t)]

    s, t = _fold_bn(params["red_bn"])
    ops += [params["red_w"].T.astype(bf16), row(s), row(t)]

    sc = params["sc"]
    s1, t1 = _fold_bn(sc["bn1"])
    s2, t2 = _fold_bn(sc["bn2"])
    ops += [row(s1), row(t1)]
    ops += [sc["w1"].T.astype(bf16), row(sc["b1"])]
    ops += [sc["w2"].T.astype(bf16), row(sc["b2"])]
    ops += [row(s2), row(t2)]

    groups_per_block = 256
    pts = 128
    n_chunks = groups_per_block * K // pts
    nblk = bm // groups_per_block

    def wspec(op):
        return pl.BlockSpec(op.shape, lambda i: (0, 0))

    out = pl.pallas_call(
        functools.partial(_gnn_kernel, n_chunks=n_chunks, pts=pts, k_sel=8),
        grid=(nblk,),
        in_specs=[
            pl.BlockSpec((groups_per_block * K, cin), lambda i: (i, 0)),
            pl.BlockSpec((3, groups_per_block * K), lambda i: (0, i)),
        ]
        + [wspec(op) for op in ops],
        out_specs=pl.BlockSpec((groups_per_block, 256), lambda i: (i, 0)),
        out_shape=jax.ShapeDtypeStruct((bm, 256), jnp.float32),
        scratch_shapes=[pltpu.VMEM((groups_per_block, 512), jnp.float32)],
    )(x, x3t, *ops)

    return out.reshape(B, M, 256).transpose(0, 2, 1)


# speedup vs baseline: 1.0034x; 1.0034x over previous
import functools

import jax
import jax.numpy as jnp
from jax.experimental import pallas as pl
from jax.experimental.pallas import tpu as pltpu

_EPS = 1e-5
_NEG = -1e30


def _fold_bn(p):
    s = p["gamma"] / jnp.sqrt(p["var"] + _EPS)
    t = p["beta"] - p["mean"] * s
    return s, t


def _dotT(a, b):
    return jax.lax.dot_general(
        a, b, (((1,), (1,)), ((), ())), preferred_element_type=jnp.float32
    )


def _dot(a, b):
    return jnp.dot(a, b, preferred_element_type=jnp.float32)


def _bdot(a, b):
    return jnp.dot(a.astype(jnp.bfloat16), b, preferred_element_type=jnp.float32)


def _gnn_kernel(
    x_ref, x3t_ref,
    e1n_ref, e1c_ref, s1e_ref, t1e_ref,
    e2n_ref, e2c_ref, s2e_ref, t2e_ref,
    ac1_ref, scb_ref, tcb_ref,
    ac2_ref, bc2_ref,
    ae_ref, se_ref, te_ref,
    ar_ref, sr_ref, tr_ref,
    s1_ref, tsc1_ref,
    w1_ref, bsc1_ref,
    w2_ref, bsc2_ref,
    s2_ref, tsc2_ref,
    out_ref,
    acc_ref,
    *, n_chunks, pts, k_sel,
):
    cp = pts
    gpc = cp // 16
    lane_i = jax.lax.broadcasted_iota(jnp.int32, (cp, cp), 1)
    lane16 = lane_i % 16
    in_group = (jax.lax.broadcasted_iota(jnp.int32, (cp, cp), 0) // 16) == (lane_i // 16)

    def load_stage(c):
        xt = x_ref[pl.ds(c * cp, cp), :]
        x3 = xt[:, 0:3]
        x3t = x3t_ref[:, pl.ds(c * cp, cp)]

        x3b = x3.astype(jnp.bfloat16)
        inner = -2.0 * _dotT(x3b, x3b)
        xx_row = jnp.sum(x3t * x3t, axis=0, keepdims=True)
        xx_col = jnp.sum(x3 * x3, axis=1, keepdims=True)
        pd = (-xx_row) - inner - xx_col
        return xt, pd

    def rank_stage(pd):
        rank = jnp.zeros((cp, cp), jnp.float32)
        for s in range(1, 16):
            wrap = lane16 + s >= 16
            rot = jnp.where(
                wrap,
                jnp.roll(pd, 16 - s, axis=1),
                jnp.roll(pd, -s, axis=1),
            )
            gt = (rot > pd) | ((rot == pd) & wrap)
            rank = rank + gt.astype(jnp.float32)
        return jnp.where(in_group, rank, 16.0)

    def edge_layer(rank, xin, wn, wc, srow, trow):
        vc = _bdot(xin, wc)
        cmx = cmn = None
        for j in range(k_sel):
            p = (rank == float(j)).astype(jnp.float32)
            g = _dot(p, xin)
            h = _bdot(g - xin, wn)
            if cmx is None:
                cmx = cmn = h
            else:
                cmx = jnp.maximum(cmx, h)
                cmn = jnp.minimum(cmn, h)
        hsel = jnp.where(srow >= 0.0, cmx + vc, cmn + vc)
        return jnp.maximum(hsel * srow + trow, 0.0)

    def tail_stage(cs, ranks, x1s):
        x2s = [
            edge_layer(rk, x1, e2n_ref[:], e2c_ref[:], s2e_ref[:], t2e_ref[:])
            for rk, x1 in zip(ranks, x1s)
        ]
        xcats = [jnp.concatenate([x1, x2], axis=1) for x1, x2 in zip(x1s, x2s)]
        c1s = [
            jnp.maximum(_bdot(xc, ac1_ref[:]) * scb_ref[:] + tcb_ref[:], 0.0)
            for xc in xcats
        ]
        c2s = [_bdot(c1, ac2_ref[:]) + bc2_ref[:] for c1 in c1s]
        xgs = [jax.nn.sigmoid(c2) * xc for c2, xc in zip(c2s, xcats)]
        es = [
            jnp.maximum(_bdot(xg, ae_ref[:]) * se_ref[:] + te_ref[:], 0.0)
            for xg in xgs
        ]
        for c, e in zip(cs, es):
            m = jnp.max(e.reshape(gpc, 16, e.shape[1]), axis=1)
            acc_ref[pl.ds(c * gpc, gpc), :] = m

    nu = 16

    def chunk_body(c, _):
        cs = [nu * c + u for u in range(nu)]
        loaded = [load_stage(ci) for ci in cs]
        ranks = [rank_stage(pd) for _, pd in loaded]
        x1s = [
            edge_layer(rk, xt, e1n_ref[:], e1c_ref[:], s1e_ref[:], t1e_ref[:])
            for rk, (xt, _) in zip(ranks, loaded)
        ]
        nt = 16
        for t in range(0, nu, nt):
            tail_stage(cs[t : t + nt], ranks[t : t + nt], x1s[t : t + nt])
        return 0

    jax.lax.fori_loop(0, n_chunks // nu, chunk_body, 0)

    av = acc_ref[:]
    r = jnp.maximum(_bdot(av, ar_ref[:]) * sr_ref[:] + tr_ref[:], 0.0)
    xb = (r + r) * s1_ref[:] + tsc1_ref[:]
    h = jnp.maximum(_bdot(xb, w1_ref[:]) + bsc1_ref[:], 0.0)
    h2 = _bdot(h, w2_ref[:]) + bsc2_ref[:]
    out_ref[:] = (xb + h2) * s2_ref[:] + tsc2_ref[:]


def kernel(xyz, feats, params):
    B, M, K, _ = xyz.shape
    bm = B * M
    x = jnp.concatenate([xyz, feats], axis=-1).reshape(bm * K, -1)
    cin = x.shape[1]
    x3t = xyz.reshape(bm * K, 3).T

    bf16 = jnp.bfloat16

    def row(v):
        return v.reshape(1, -1)

    ops = []

    def edge_prep(lyr):
        w = lyr["w"]
        c = w.shape[1] // 2
        s, t = _fold_bn(lyr["bn"])
        wn = w[:, :c].T.astype(bf16)
        wc = w[:, c:].T.astype(bf16)
        return wn, wc, row(s), row(t)

    ops += list(edge_prep(params["edge"][0]))
    ops += list(edge_prep(params["edge"][1]))

    s, t = _fold_bn(params["calib_bn"])
    ops += [params["calib_w1"].T.astype(bf16), row(s), row(t)]
    ops += [params["calib_w2"].T.astype(bf16), row(params["calib_b2"])]

    s, t = _fold_bn(params["exp_bn"])
    ops += [params["exp_w"].T.astype(bf16), row(s), row(t)]

    s, t = _fold_bn(params["red_bn"])
    ops += [params["red_w"].T.astype(bf16), row(s), row(t)]

    sc = params["sc"]
    s1, t1 = _fold_bn(sc["bn1"])
    s2, t2 = _fold_bn(sc["bn2"])
    ops += [row(s1), row(t1)]
    ops += [sc["w1"].T.astype(bf16), row(sc["b1"])]
    ops += [sc["w2"].T.astype(bf16), row(sc["b2"])]
    ops += [row(s2), row(t2)]

    groups_per_block = 128
    pts = 128
    n_chunks = groups_per_block * K // pts
    nblk = bm // groups_per_block

    def wspec(op):
        return pl.BlockSpec(op.shape, lambda i: (0, 0))

    out = pl.pallas_call(
        functools.partial(_gnn_kernel, n_chunks=n_chunks, pts=pts, k_sel=8),
        grid=(nblk,),
        in_specs=[
            pl.BlockSpec((groups_per_block * K, cin), lambda i: (i, 0)),
            pl.BlockSpec((3, groups_per_block * K), lambda i: (0, i)),
        ]
        + [wspec(op) for op in ops],
        out_specs=pl.BlockSpec((groups_per_block, 256), lambda i: (i, 0)),
        out_shape=jax.ShapeDtypeStruct((bm, 256), jnp.float32),
        scratch_shapes=[pltpu.VMEM((groups_per_block, 512), jnp.float32)],
    )(x, x3t, *ops)

    return out.reshape(B, M, 256).transpose(0, 2, 1)
